# baseline (speedup 1.0000x reference)
import functools

import jax
import jax.numpy as jnp
from jax import lax
from jax.experimental import pallas as pl
from jax.experimental.pallas import tpu as pltpu
from jax.experimental.pallas import tpu_sc as plsc

_NS = 16
_NC = 2
_CH = 128


def _matmul(x, w):
    n = x.shape[0]
    d = w.shape[1]

    def body(x_ref, w_ref, o_ref):
        o_ref[...] = jnp.dot(x_ref[...], w_ref[...],
                             preferred_element_type=jnp.float32)

    return pl.pallas_call(
        body,
        out_shape=jax.ShapeDtypeStruct((n, d), jnp.float32),
    )(x, w)


def _combine(p, b):
    _, n, d = p.shape
    blk = 2000

    def body(p_ref, b_ref, o_ref):
        o_ref[...] = p_ref[0] + p_ref[1] + b_ref[...]

    return pl.pallas_call(
        body,
        grid=(n // blk,),
        in_specs=[
            pl.BlockSpec((_NC, blk, d), lambda i: (0, i, 0)),
            pl.BlockSpec((1, d), lambda i: (0, 0)),
        ],
        out_specs=pl.BlockSpec((blk, d), lambda i: (i, 0)),
        out_shape=jax.ShapeDtypeStruct((n, d), jnp.float32),
    )(p, b.reshape(1, d))


def _row_chunks(total, step):
    sizes = []
    left = total
    while left > 0:
        sizes.append(min(step, left))
        left -= sizes[-1]
    return sizes


def _spmm_sc(src1, dst2, ew1, sup):
    n, d = sup.shape
    nw = _NC * _NS
    ncw = dst2.shape[0] // nw
    npass = 2
    ncp = ncw // npass
    r_tile = (n // _NS) // 8 * 8
    r_last = n - (_NS - 1) * r_tile
    nvec = d // 16

    mesh = plsc.VectorSubcoreMesh(core_axis_name="c", subcore_axis_name="s")

    @functools.partial(
        pl.kernel,
        out_type=jax.ShapeDtypeStruct((_NC, n, d), jnp.float32),
        mesh=mesh,
        scratch_types=[
            pltpu.VMEM((ncp * _CH,), jnp.int32),
            pltpu.VMEM((ncp, _CH), jnp.int32),
            pltpu.VMEM((ncp * _CH,), jnp.float32),
            pltpu.VMEM((_CH, d), jnp.float32),
            pltpu.VMEM((_CH, d), jnp.float32),
            pltpu.VMEM_SHARED((n, d), jnp.float32),
            pltpu.SemaphoreType.DMA,
            pltpu.SemaphoreType.DMA,
            pltpu.SemaphoreType.DMA,
            pltpu.SemaphoreType.DMA,
        ],
    )
    def spmm(src_h, dst_h, ew_h, sup_h, out_h, sidx, didx, wv, rows_a,
             rows_b, acc, gs_a, gs_b, ss_a, ss_b):
        c = lax.axis_index("c")
        s = lax.axis_index("s")
        wid = s * _NC + c

        def gather_start(k, buf, sem):
            pltpu.async_copy(sup_h.at[sidx.at[pl.ds(k * _CH, _CH)]], buf, sem)

        def gather_wait(k, buf, sem):
            pltpu.make_async_copy(sup_h.at[sidx.at[pl.ds(k * _CH, _CH)]],
                                  buf, sem).wait()

        def scatter_start(k, buf, sem):
            pltpu.async_copy(buf, acc.at[didx.at[k]], sem, add=True)

        def scatter_wait(k, buf, sem):
            pltpu.make_async_copy(buf, acc.at[didx.at[k]], sem).wait()

        def scale(k, buf):
            def body(g, carry):
                w16 = wv[pl.ds(k * _CH + g * 16, 16)]
                for t in range(16):
                    w = w16[t]
                    i = g * 16 + t
                    for j in range(nvec):
                        sl = (i, pl.ds(16 * j, 16))
                        buf[sl] = buf[sl] * w
                return carry
            lax.fori_loop(0, _CH // 16, body, 0)

        def for_slab(fn):
            @pl.when(s < _NS - 1)
            def _():
                fn(s * r_tile, _row_chunks(r_tile, _CH))

            @pl.when(s == _NS - 1)
            def _():
                fn((_NS - 1) * r_tile, _row_chunks(r_last, _CH))

        zero = jnp.zeros((16,), jnp.float32)

        def zbody(i, carry):
            for j in range(nvec):
                rows_a[i, pl.ds(16 * j, 16)] = zero
            return carry
        lax.fori_loop(0, _CH, zbody, 0)

        def init_fn(r0, sizes):
            off = 0
            for sz in sizes:
                base = pl.multiple_of(r0 + off, 8)
                pltpu.sync_copy(rows_a.at[pl.ds(0, sz)],
                                acc.at[pl.ds(base, sz)])
                off += sz
        for_slab(init_fn)
        plsc.subcore_barrier()

        for p in range(npass):
            c0 = pl.multiple_of(wid * ncw + p * ncp, 8)
            e0 = pl.multiple_of((wid * ncw + p * ncp) * _CH, 8)
            pltpu.sync_copy(src_h.at[pl.ds(e0, ncp * _CH)], sidx)
            pltpu.sync_copy(dst_h.at[pl.ds(c0, ncp)], didx)
            pltpu.sync_copy(ew_h.at[pl.ds(e0, ncp * _CH)], wv)

            gather_start(0, rows_a, gs_a)
            gather_start(1, rows_b, gs_b)

            def ebody(kk, carry):
                a = kk * 2
                b = a + 1
                gather_wait(a, rows_a, gs_a)
                scale(a, rows_a)
                scatter_start(a, rows_a, ss_a)
                gather_wait(b, rows_b, gs_b)
                scale(b, rows_b)
                scatter_start(b, rows_b, ss_b)
                scatter_wait(a, rows_a, ss_a)
                gather_start(a + 2, rows_a, gs_a)
                scatter_wait(b, rows_b, ss_b)
                gather_start(b + 2, rows_b, gs_b)
                return carry
            lax.fori_loop(0, ncp // 2 - 1, ebody, 0)

            a = ncp - 2
            b = ncp - 1
            gather_wait(a, rows_a, gs_a)
            scale(a, rows_a)
            scatter_start(a, rows_a, ss_a)
            gather_wait(b, rows_b, gs_b)
            scale(b, rows_b)
            scatter_start(b, rows_b, ss_b)
            scatter_wait(a, rows_a, ss_a)
            scatter_wait(b, rows_b, ss_b)
        plsc.subcore_barrier()

        def out_fn(r0, sizes):
            off = 0
            for sz in sizes:
                base = pl.multiple_of(r0 + off, 8)
                pltpu.sync_copy(acc.at[pl.ds(base, sz)],
                                rows_a.at[pl.ds(0, sz)])
                pltpu.sync_copy(rows_a.at[pl.ds(0, sz)],
                                out_h.at[c, pl.ds(base, sz)])
                off += sz
        for_slab(out_fn)

    return spmm(src1, dst2, ew1, sup)


def kernel(edge_index, edge_weight, input_feature, W, b):
    src = edge_index[0]
    dst = edge_index[1]
    e = src.shape[0]
    nw = _NC * _NS
    ncw = -(-e // (nw * _CH))
    ncw = -(-ncw // 8) * 8
    ep = nw * ncw * _CH
    pad = ep - e
    src1 = jnp.pad(src, (0, pad))
    dst2 = jnp.pad(dst, (0, pad)).reshape(ep // _CH, _CH)
    ew1 = jnp.pad(edge_weight, (0, pad))
    sup = _matmul(input_feature, W)
    partials = _spmm_sc(src1, dst2, ew1, sup)
    return _combine(partials, b)
